# baseline (speedup 1.0000x reference)
import jax
import jax.numpy as jnp
from jax import lax
from jax.experimental import pallas as pl
from jax.experimental.pallas import tpu as pltpu
from jax.experimental.pallas import tpu_sc as plsc

_B, _T, _D = 4, 8192, 768

_INFO = plsc.get_sparse_core_info()
_NC = _INFO.num_cores
_NS = _INFO.num_subcores
_NW = _NC * _NS
_ROWS = _T // _NW


def _sc_body(table_hbm, out_hbm, sem):
    wid = lax.axis_index("s") * _NC + lax.axis_index("c")
    base = wid * _ROWS
    copies = [
        pltpu.async_copy(
            table_hbm.at[pl.ds(base, _ROWS)],
            out_hbm.at[b, pl.ds(base, _ROWS)],
            sem,
        )
        for b in range(_B)
    ]
    for h in copies:
        h.wait()


def kernel(x, table):
    del x
    mesh = plsc.VectorSubcoreMesh(core_axis_name="c", subcore_axis_name="s")
    run = pl.kernel(
        _sc_body,
        mesh=mesh,
        out_type=jax.ShapeDtypeStruct((_B, _T, _D), jnp.float32),
        scratch_types=[pltpu.SemaphoreType.DMA],
    )
    return run(table)


# speedup vs baseline: 51.7931x; 51.7931x over previous
import jax
import jax.numpy as jnp
from jax import lax
from jax.experimental import pallas as pl
from jax.experimental.pallas import tpu as pltpu
from jax.experimental.pallas import tpu_sc as plsc
from jax._src.pallas import mpmd as pl_mpmd

_B, _T, _D = 4, 8192, 768

_INFO = plsc.get_sparse_core_info()
_NC = _INFO.num_cores
_NS = _INFO.num_subcores
_NW = _NC * _NS

_RT = 160
_TEC_CHUNK = 80
_TEC_NCHUNK = _RT // _TEC_CHUNK
_TEC_ROWS = _NW * _RT

_SS = (_T - _TEC_ROWS) // _NC
_SCS_CHUNK = 256
_SCS_NCHUNK = _SS // _SCS_CHUNK


def _tec_fn(table, out, tbuf, sb0, sb1, sem0, sem1):
    del sb0, sb1, sem0, sem1
    wid = lax.axis_index("s") * _NC + lax.axis_index("c")
    base = wid * _RT
    for ch in range(_TEC_NCHUNK):
        row0 = base + ch * _TEC_CHUNK
        pltpu.sync_copy(table.at[pl.ds(row0, _TEC_CHUNK)], tbuf)
        for b in range(_B):
            pltpu.sync_copy(tbuf, out.at[b, pl.ds(row0, _TEC_CHUNK)])


def _scs_fn(table, out, tbuf, sb0, sb1, sem0, sem1):
    del tbuf
    cid = lax.axis_index("c")
    base = _TEC_ROWS + cid * _SS
    bufs = (sb0, sb1)
    sems = (sem0, sem1)
    reads = [None, None]
    writes = [None, None]
    reads[0] = pltpu.async_copy(table.at[pl.ds(base, _SCS_CHUNK)], bufs[0], sems[0])
    for ch in range(_SCS_NCHUNK):
        i = ch % 2
        ni = (ch + 1) % 2
        if ch + 1 < _SCS_NCHUNK:
            if writes[ni] is not None:
                for h in writes[ni]:
                    h.wait()
                writes[ni] = None
            reads[ni] = pltpu.async_copy(
                table.at[pl.ds(base + (ch + 1) * _SCS_CHUNK, _SCS_CHUNK)],
                bufs[ni],
                sems[ni],
            )
        reads[i].wait()
        row0 = base + ch * _SCS_CHUNK
        writes[i] = [
            pltpu.async_copy(bufs[i], out.at[b, pl.ds(row0, _SCS_CHUNK)], sems[i])
            for b in range(_B)
        ]
    for group in writes:
        if group is not None:
            for h in group:
                h.wait()


def kernel(x, table):
    del x
    scalar_mesh = plsc.ScalarSubcoreMesh(axis_name="c")
    vector_mesh = plsc.VectorSubcoreMesh(core_axis_name="c", subcore_axis_name="s")
    run = pl_mpmd.mpmd_map(
        [(scalar_mesh, _scs_fn), (vector_mesh, _tec_fn)],
        out_types=[jax.ShapeDtypeStruct((_B, _T, _D), jnp.float32)],
        scratch_types=[
            (pltpu.VMEM @ vector_mesh)((_TEC_CHUNK, _D), jnp.float32),
            pltpu.VMEM_SHARED((_SCS_CHUNK, _D), jnp.float32),
            pltpu.VMEM_SHARED((_SCS_CHUNK, _D), jnp.float32),
            pltpu.SemaphoreType.DMA @ scalar_mesh,
            pltpu.SemaphoreType.DMA @ scalar_mesh,
        ],
    )
    [out] = run(table)
    return out


# speedup vs baseline: 51.9081x; 1.0022x over previous
import jax
import jax.numpy as jnp
from jax import lax
from jax.experimental import pallas as pl
from jax.experimental.pallas import tpu as pltpu
from jax.experimental.pallas import tpu_sc as plsc

_B, _T, _D = 4, 8192, 768

_INFO = plsc.get_sparse_core_info()
_NC = _INFO.num_cores
_NS = _INFO.num_subcores
_NW = _NC * _NS
_ROWS = _T // _NW
_CHUNK = 128
_NCHUNK = _ROWS // _CHUNK


def _sc_body(table_hbm, out_hbm, buf):
    wid = lax.axis_index("s") * _NC + lax.axis_index("c")
    base = wid * _ROWS
    for ch in range(_NCHUNK):
        row0 = base + ch * _CHUNK
        pltpu.sync_copy(table_hbm.at[pl.ds(row0, _CHUNK)], buf)
        for b in range(_B):
            pltpu.sync_copy(buf, out_hbm.at[b, pl.ds(row0, _CHUNK)])


def kernel(x, table):
    del x
    mesh = plsc.VectorSubcoreMesh(core_axis_name="c", subcore_axis_name="s")
    run = pl.kernel(
        _sc_body,
        mesh=mesh,
        out_type=jax.ShapeDtypeStruct((_B, _T, _D), jnp.float32),
        scratch_types=[pltpu.VMEM((_CHUNK, _D), jnp.float32)],
    )
    return run(table)
